# baseline (speedup 1.0000x reference)
import functools

import jax
import jax.numpy as jnp
from jax import lax
from jax.experimental import pallas as pl
from jax.experimental.pallas import tpu as pltpu
from jax.experimental.pallas import tpu_sc as plsc

VOCAB = 1000000
DIM = 64
BATCH = 4096
WIDTH = 26
BW = BATCH * WIDTH
NC, NS, L = 2, 16, 16
NW = NC * NS
PER_W = BW // NW
G = 128
NG = PER_W // G

_mesh = plsc.VectorSubcoreMesh(core_axis_name="c", subcore_axis_name="s")


@functools.partial(
    pl.kernel,
    mesh=_mesh,
    out_type=jax.ShapeDtypeStruct((BW, DIM), jnp.float32),
    scratch_types=[
        pltpu.VMEM((NG, G), jnp.int32),
        pltpu.VMEM((G, DIM), jnp.float32),
        pltpu.SemaphoreType.DMA,
    ],
    compiler_params=pltpu.CompilerParams(
        use_tc_tiling_on_sc=False, needs_layout_passes=False),
)
def _emb_lookup(idx_hbm, table_hbm, out_hbm, idx_v, rows_v, gsem):
    wid = lax.axis_index("s") * NC + lax.axis_index("c")

    pltpu.sync_copy(idx_hbm.at[pl.ds(wid * NG, NG)], idx_v)

    def block(j, carry):
        pltpu.async_copy(table_hbm.at[idx_v.at[j]], rows_v, gsem).wait()

        def group(g, carry2):
            iv = idx_v[j, pl.ds(g * L, L)]
            m = iv == 0
            ones = jnp.where(m, jnp.ones((L,), jnp.int32),
                             jnp.zeros((L,), jnp.int32))
            nzero = jnp.sum(ones)

            @pl.when(nzero > 0)
            def _():
                rows = lax.iota(jnp.int32, L) + g * L
                zeros = jnp.zeros((L,), jnp.float32)

                def col(c, carry3):
                    cols = jnp.full((L,), c, jnp.int32)
                    plsc.store_scatter(rows_v, [rows, cols], zeros, mask=m)
                    return carry3

                lax.fori_loop(0, DIM, col, 0)

            return carry2

        lax.fori_loop(0, G // L, group, 0)

        pltpu.sync_copy(rows_v,
                        out_hbm.at[pl.ds(wid * PER_W + j * G, G)])
        return carry

    lax.fori_loop(0, NG, block, 0)


def kernel(input, table):
    idx = jnp.maximum(input.astype(jnp.int32), 0).reshape(NW * NG, G)
    out = _emb_lookup(idx, table)
    vals = out.reshape(BATCH, WIDTH, DIM)
    return jnp.where((input != 0)[..., None], vals, jnp.float32(0.0))


# speedup vs baseline: 1.0596x; 1.0596x over previous
import functools

import jax
import jax.numpy as jnp
from jax import lax
from jax.experimental import pallas as pl
from jax.experimental.pallas import tpu as pltpu
from jax.experimental.pallas import tpu_sc as plsc

VOCAB = 1000000
DIM = 64
BATCH = 4096
WIDTH = 26
BW = BATCH * WIDTH
NC, NS, L = 2, 16, 16
NW = NC * NS
PER_W = BW // NW
G = 128
NG = PER_W // G

_mesh = plsc.VectorSubcoreMesh(core_axis_name="c", subcore_axis_name="s")


@functools.partial(
    pl.kernel,
    mesh=_mesh,
    out_type=jax.ShapeDtypeStruct((BW, DIM), jnp.float32),
    scratch_types=[
        pltpu.VMEM((NG, G), jnp.int32),
        pltpu.VMEM((G, DIM), jnp.float32),
        pltpu.SemaphoreType.DMA,
    ],
    compiler_params=pltpu.CompilerParams(
        use_tc_tiling_on_sc=False, needs_layout_passes=False),
)
def _emb_lookup(idx_hbm, table_hbm, out_hbm, idx_v, rows_v, gsem):
    wid = lax.axis_index("s") * NC + lax.axis_index("c")

    pltpu.sync_copy(idx_hbm.at[pl.ds(wid * NG, NG)], idx_v)

    def block(j, carry):
        pltpu.async_copy(table_hbm.at[idx_v.at[j]], rows_v, gsem).wait()

        def group(g, carry2):
            iv = idx_v[j, pl.ds(g * L, L)]
            m = iv == 0
            ones = jnp.where(m, jnp.ones((L,), jnp.int32),
                             jnp.zeros((L,), jnp.int32))
            nzero = jnp.sum(ones)

            @pl.when(nzero > 0)
            def _():
                rows = lax.iota(jnp.int32, L) + g * L
                zeros = jnp.zeros((L,), jnp.float32)

                def col(c, carry3):
                    cols = jnp.full((L,), c, jnp.int32)
                    plsc.store_scatter(rows_v, [rows, cols], zeros, mask=m)
                    return carry3

                lax.fori_loop(0, DIM, col, 0)

            return carry2

        lax.fori_loop(0, G // L, group, 0)

        pltpu.sync_copy(rows_v,
                        out_hbm.at[pl.ds(wid * PER_W + j * G, G)])
        return carry

    lax.fori_loop(0, NG, block, 0)


def kernel(input, table):
    idx = jnp.maximum(input.astype(jnp.int32), 0).reshape(NW * NG, G)
    out = _emb_lookup(idx, table)
    return out.reshape(BATCH, WIDTH, DIM)


# speedup vs baseline: 1.0850x; 1.0240x over previous
import functools

import jax
import jax.numpy as jnp
from jax import lax
from jax.experimental import pallas as pl
from jax.experimental.pallas import tpu as pltpu
from jax.experimental.pallas import tpu_sc as plsc

VOCAB = 1000000
DIM = 64
BATCH = 4096
WIDTH = 26
BW = BATCH * WIDTH
NC, NS, L = 2, 16, 16
NW = NC * NS
PER_W = BW // NW
G = 128
NG = PER_W // G

_mesh = plsc.VectorSubcoreMesh(core_axis_name="c", subcore_axis_name="s")


@functools.partial(
    pl.kernel,
    mesh=_mesh,
    out_type=jax.ShapeDtypeStruct((BW, DIM), jnp.float32),
    scratch_types=[
        pltpu.VMEM((NG, G), jnp.int32),
        pltpu.VMEM((G, DIM), jnp.float32),
        pltpu.VMEM((G, DIM), jnp.float32),
        pltpu.SemaphoreType.DMA,
        pltpu.SemaphoreType.DMA,
        pltpu.SemaphoreType.DMA,
        pltpu.SemaphoreType.DMA,
    ],
    compiler_params=pltpu.CompilerParams(
        use_tc_tiling_on_sc=False, needs_layout_passes=False),
)
def _emb_lookup(idx_hbm, table_hbm, out_hbm, idx_v, rows_v0, rows_v1,
                gsem0, gsem1, osem0, osem1):
    wid = lax.axis_index("s") * NC + lax.axis_index("c")
    bufs = (rows_v0, rows_v1)
    gsems = (gsem0, gsem1)
    osems = (osem0, osem1)

    pltpu.sync_copy(idx_hbm.at[pl.ds(wid * NG, NG)], idx_v)

    def mask_zero_rows(buf, j):
        def group(g, carry2):
            iv = idx_v[j, pl.ds(g * L, L)]
            m = iv == 0
            ones = jnp.where(m, jnp.ones((L,), jnp.int32),
                             jnp.zeros((L,), jnp.int32))
            nzero = jnp.sum(ones)

            @pl.when(nzero > 0)
            def _():
                rows = lax.iota(jnp.int32, L) + g * L
                zeros = jnp.zeros((L,), jnp.float32)

                def col(c, carry3):
                    cols = jnp.full((L,), c, jnp.int32)
                    plsc.store_scatter(buf, [rows, cols], zeros, mask=m)
                    return carry3

                lax.fori_loop(0, DIM, col, 0)

            return carry2

        lax.fori_loop(0, G // L, group, 0)

    for b in range(2):
        pltpu.async_copy(table_hbm.at[idx_v.at[b]], bufs[b], gsems[b])

    def step(j, carry):
        for b in range(2):
            jj = 2 * j + b
            buf, gsem, osem = bufs[b], gsems[b], osems[b]
            pltpu.make_async_copy(table_hbm.at[idx_v.at[jj]], buf,
                                  gsem).wait()
            mask_zero_rows(buf, jj)
            dst = out_hbm.at[pl.ds(wid * PER_W + jj * G, G)]
            pltpu.async_copy(buf, dst, osem)
            pltpu.make_async_copy(buf, dst, osem).wait()

            @pl.when(jj + 2 < NG)
            def _():
                pltpu.async_copy(table_hbm.at[idx_v.at[jj + 2]], buf, gsem)

        return carry

    lax.fori_loop(0, NG // 2, step, 0)


def kernel(input, table):
    idx = jnp.maximum(input.astype(jnp.int32), 0).reshape(NW * NG, G)
    out = _emb_lookup(idx, table)
    return out.reshape(BATCH, WIDTH, DIM)
